# baseline (speedup 1.0000x reference)
import functools

import jax
import jax.numpy as jnp
from jax import lax
from jax.experimental import pallas as pl
from jax.experimental.pallas import tpu as pltpu
from jax.experimental.pallas import tpu_sc as plsc

N = 10000
NP = 10240
E = 320000
EP = 327680
WORKERS = 32
BATCH = 128
TOTB = EP // BATCH
NB0 = 160
NB1 = 0
S = 8
ROWS_PER_TILE = NP // 16
GATHER_ONLY = False
SCATTER_ONLY = False


def _seg_sum_call(x3, src3, dst3, C, F):
    mesh = plsc.VectorSubcoreMesh(core_axis_name="c", subcore_axis_name="s")

    @functools.partial(
        pl.kernel,
        mesh=mesh,
        out_type=jax.ShapeDtypeStruct((2, C, NP, F), jnp.float32),
        scratch_types=[
            pltpu.VMEM((2, S, BATCH), jnp.int32),
            pltpu.VMEM((2, S, BATCH), jnp.int32),
            pltpu.VMEM((2 * BATCH, F), jnp.float32),
            pltpu.VMEM_SHARED((NP, F), jnp.float32),
            pltpu.SemaphoreType.DMA,
            pltpu.SemaphoreType.DMA,
            pltpu.SemaphoreType.DMA,
            pltpu.SemaphoreType.DMA,
            pltpu.SemaphoreType.DMA,
        ],
    )
    def seg_kernel(x3_hbm, src_hbm, dst_hbm, out_hbm,
                   isrc, idst, rows, agg_sh, sem_i, sg0, sg1, ss0, ss1):
        core = lax.axis_index("c")
        sub = lax.axis_index("s")
        row0 = sub * ROWS_PER_TILE
        sg = (sg0, sg1)
        ss = (ss0, ss1)
        base = jnp.where(core == 0, sub * NB0, 16 * NB0 + sub * NB1)
        nsb = jnp.where(core == 0, NB0 // S, NB1 // S)

        def slot(p):
            return rows.at[pl.ds(p * BATCH, BATCH)]

        def idx_desc(q, sb):
            return (pltpu.make_async_copy(
                        src_hbm.at[pl.ds(base + sb * S, S)], isrc.at[q], sem_i),
                    pltpu.make_async_copy(
                        dst_hbm.at[pl.ds(base + sb * S, S)], idst.at[q], sem_i))

        def g_desc(c, q, j, p):
            return pltpu.make_async_copy(
                x3_hbm.at[c].at[isrc.at[q, j]], slot(p), sg[p])

        def s_desc(q, j, p):
            return pltpu.make_async_copy(
                slot(p), agg_sh.at[idst.at[q, j]], ss[p])

        def chunk_body(c, _):
            def zrow(r, _):
                def zcol(k2, _):
                    rows[r, pl.ds(k2 * 16, 16)] = jnp.zeros((16,), jnp.float32)
                    return 0
                return lax.fori_loop(0, F // 16, zcol, 0)
            lax.fori_loop(0, 64, zrow, 0)
            zsrc = rows.at[pl.ds(0, 64)]
            for i in range(ROWS_PER_TILE // 64):
                pltpu.async_copy(zsrc, agg_sh.at[pl.ds(row0 + i * 64, 64)], ss0)
            for i in range(ROWS_PER_TILE // 64):
                pltpu.make_async_copy(
                    zsrc, agg_sh.at[pl.ds(row0 + i * 64, 64)], ss0).wait()
            plsc.subcore_barrier()

            @pl.when(nsb > 0)
            def _():
                for d in idx_desc(0, 0):
                    d.start()

            def sb_body(sb, _):
                q = sb % 2
                for j in range(S):
                    p = j % 2
                    if not GATHER_ONLY:
                        if j >= 2:
                            s_desc(q, j - 2, p).wait()
                        else:
                            @pl.when(sb > 0)
                            def _():
                                s_desc(1 - q, j + S - 2, p).wait()
                    if j == 0:
                        for d in idx_desc(q, sb):
                            d.wait()
                    if not SCATTER_ONLY:
                        g_desc(c, q, j, p).start()
                    if j >= 1:
                        if not SCATTER_ONLY:
                            g_desc(c, q, j - 1, 1 - p).wait()
                        if not GATHER_ONLY:
                            pltpu.async_copy(
                                slot(1 - p), agg_sh.at[idst.at[q, j - 1]],
                                ss[1 - p], add=True)
                    else:
                        @pl.when(sb > 0)
                        def _():
                            if not SCATTER_ONLY:
                                g_desc(c, 1 - q, S - 1, 1 - p).wait()
                            if not GATHER_ONLY:
                                pltpu.async_copy(
                                    slot(1 - p), agg_sh.at[idst.at[1 - q, S - 1]],
                                    ss[1 - p], add=True)
                        @pl.when(sb + 1 < nsb)
                        def _():
                            for d in idx_desc(1 - q, sb + 1):
                                d.start()
                return 0
            lax.fori_loop(0, nsb, sb_body, 0)

            @pl.when(nsb > 0)
            def _():
                qL = (nsb - 1) % 2
                pL = (S - 1) % 2
                if not SCATTER_ONLY:
                    g_desc(c, qL, S - 1, pL).wait()
                if not GATHER_ONLY:
                    pltpu.async_copy(slot(pL), agg_sh.at[idst.at[qL, S - 1]],
                                     ss[pL], add=True)
                    s_desc(qL, S - 2, 1 - pL).wait()
                    s_desc(qL, S - 1, pL).wait()
            plsc.subcore_barrier()

            pltpu.sync_copy(agg_sh.at[pl.ds(row0, ROWS_PER_TILE)],
                            out_hbm.at[core].at[c].at[pl.ds(row0, ROWS_PER_TILE)])
            plsc.subcore_barrier()
            return 0

        lax.fori_loop(0, C, chunk_body, 0)

    return seg_kernel(x3, src3, dst3)


RB = 256


def _mlp_call(x3, agg3, W1, b1, W2, b2, relu_out, out_chunks):
    C, _, F = x3.shape
    D, H = W1.shape
    H2 = W2.shape[1]
    C2 = out_chunks

    def body(x_ref, a_ref, w1_ref, b1_ref, w2_ref, b2_ref, o_ref):
        xb = jnp.concatenate(
            [x_ref[c] + a_ref[0, c] + a_ref[1, c] for c in range(C)], axis=1)
        h = jnp.dot(xb, w1_ref[...], preferred_element_type=jnp.float32)
        h = jnp.maximum(h + b1_ref[0], 0.0)
        y = jnp.dot(h, w2_ref[...], preferred_element_type=jnp.float32)
        y = y + b2_ref[0]
        if relu_out:
            y = jnp.maximum(y, 0.0)
        if C2 is None:
            o_ref[...] = y
        else:
            for c2 in range(C2):
                o_ref[c2] = y[:, c2 * 128:(c2 + 1) * 128]

    if C2 is None:
        out_shape = jax.ShapeDtypeStruct((NP, H2), jnp.float32)
        out_spec = pl.BlockSpec((RB, H2), lambda i: (i, 0))
    else:
        out_shape = jax.ShapeDtypeStruct((C2, NP, 128), jnp.float32)
        out_spec = pl.BlockSpec((C2, RB, 128), lambda i: (0, i, 0))

    return pl.pallas_call(
        body,
        grid=(NP // RB,),
        in_specs=[
            pl.BlockSpec((C, RB, F), lambda i: (0, i, 0)),
            pl.BlockSpec((2, C, RB, F), lambda i: (0, 0, i, 0)),
            pl.BlockSpec((D, H), lambda i: (0, 0)),
            pl.BlockSpec((1, H), lambda i: (0, 0)),
            pl.BlockSpec((H, H2), lambda i: (0, 0)),
            pl.BlockSpec((1, H2), lambda i: (0, 0)),
        ],
        out_specs=out_spec,
        out_shape=out_shape,
    )(x3, agg3, W1, b1.reshape(1, H), W2, b2.reshape(1, H2))


def _proj_call(x3, W, out_chunks):
    C, _, F = x3.shape
    D, H2 = W.shape
    C2 = out_chunks

    def body(x_ref, w_ref, o_ref):
        xb = jnp.concatenate([x_ref[c] for c in range(C)], axis=1)
        y = jnp.dot(xb, w_ref[...], preferred_element_type=jnp.float32)
        for c2 in range(C2):
            o_ref[c2] = y[:, c2 * 128:(c2 + 1) * 128]

    return pl.pallas_call(
        body,
        grid=(NP // RB,),
        in_specs=[
            pl.BlockSpec((C, RB, F), lambda i: (0, i, 0)),
            pl.BlockSpec((D, H2), lambda i: (0, 0)),
        ],
        out_specs=pl.BlockSpec((C2, RB, 128), lambda i: (0, i, 0)),
        out_shape=jax.ShapeDtypeStruct((C2, NP, 128), jnp.float32),
    )(x3, W)


def _gin_tail_call(u3, agg3, b1, W2, b2):
    C, _, F = u3.shape
    H = C * F
    H2 = W2.shape[1]

    def body(u_ref, a_ref, b1_ref, w2_ref, b2_ref, o_ref):
        h = jnp.concatenate(
            [u_ref[c] + a_ref[0, c] + a_ref[1, c] for c in range(C)], axis=1)
        h = jnp.maximum(h + b1_ref[0], 0.0)
        o_ref[...] = jnp.dot(
            h, w2_ref[...], preferred_element_type=jnp.float32) + b2_ref[0]

    return pl.pallas_call(
        body,
        grid=(NP // RB,),
        in_specs=[
            pl.BlockSpec((C, RB, F), lambda i: (0, i, 0)),
            pl.BlockSpec((2, C, RB, F), lambda i: (0, 0, i, 0)),
            pl.BlockSpec((1, H), lambda i: (0, 0)),
            pl.BlockSpec((H, H2), lambda i: (0, 0)),
            pl.BlockSpec((1, H2), lambda i: (0, 0)),
        ],
        out_specs=pl.BlockSpec((RB, H2), lambda i: (i, 0)),
        out_shape=jax.ShapeDtypeStruct((NP, H2), jnp.float32),
    )(u3, agg3, b1.reshape(1, H), W2, b2.reshape(1, H2))


def _heads_call(x, Wcat, bcat, eps):
    H2 = x.shape[1]

    def body(x_ref, w_ref, b_ref, e_ref, z_ref, m_ref, v_ref):
        y = jnp.dot(x_ref[...], w_ref[...],
                    preferred_element_type=jnp.float32) + b_ref[0]
        m = y[:, :128]
        v = y[:, 128:]
        m_ref[...] = m
        v_ref[...] = v
        z_ref[...] = m + v * e_ref[...]

    out_shape = [jax.ShapeDtypeStruct((NP, 128), jnp.float32)] * 3
    return pl.pallas_call(
        body,
        grid=(NP // RB,),
        in_specs=[
            pl.BlockSpec((RB, H2), lambda i: (i, 0)),
            pl.BlockSpec((H2, 256), lambda i: (0, 0)),
            pl.BlockSpec((1, 256), lambda i: (0, 0)),
            pl.BlockSpec((RB, 128), lambda i: (i, 0)),
        ],
        out_specs=[pl.BlockSpec((RB, 128), lambda i: (i, 0))] * 3,
        out_shape=out_shape,
    )(x, Wcat, bcat.reshape(1, 256), eps)


def _to_chunks(x, C, F):
    return jnp.transpose(x.reshape(NP, C, F), (1, 0, 2))


def kernel(eeg_nodes, eeg_idx,
           W1_0, b1_0, W2_0, b2_0,
           W1_1, b1_1, W2_1, b2_1,
           W1_2, b1_2, W2_2, b2_2,
           W1_3, b1_3, W2_3, b2_3,
           Wm, bm, Wv, bv):
    src = eeg_idx[0]
    dst = eeg_idx[1]
    pad_e = EP - E
    src_p = jnp.concatenate([src, jnp.zeros((pad_e,), jnp.int32)])
    dst_p = jnp.concatenate([dst, jnp.full((pad_e,), N, jnp.int32)])
    src3 = src_p.reshape(TOTB, BATCH)
    dst3 = dst_p.reshape(TOTB, BATCH)

    x0 = jnp.pad(eeg_nodes, ((0, NP - N), (0, 0)))
    x3_0 = x0[None]

    agg0 = _seg_sum_call(x3_0, src3, dst3, C=1, F=128)
    x1 = _mlp_call(x3_0, agg0, W1_0, b1_0, W2_0, b2_0,
                   relu_out=True, out_chunks=16)

    agg1 = _seg_sum_call(x1, src3, dst3, C=16, F=128)
    x2 = _mlp_call(x1, agg1, W1_1, b1_1, W2_1, b2_1,
                   relu_out=True, out_chunks=16)

    agg2 = _seg_sum_call(x2, src3, dst3, C=16, F=128)
    x3_ = _mlp_call(x2, agg2, W1_2, b1_2, W2_2, b2_2,
                    relu_out=True, out_chunks=16)

    u3 = _proj_call(x3_, W1_3, out_chunks=8)
    agg3 = _seg_sum_call(u3, src3, dst3, C=8, F=128)
    x4 = _gin_tail_call(u3, agg3, b1_3, W2_3, b2_3)

    Wcat = jnp.concatenate([Wm, Wv], axis=1)
    bcat = jnp.concatenate([bm, bv])
    epsn = jax.random.normal(jax.random.key(1234), (N, 128), jnp.float32)
    eps_p = jnp.pad(epsn, ((0, NP - N), (0, 0)))
    z, mean, var = _heads_call(x4, Wcat, bcat, eps_p)
    return (z[:N], mean[:N], var[:N])


# speedup vs baseline: 1.3939x; 1.3939x over previous
import functools

import jax
import jax.numpy as jnp
from jax import lax
from jax.experimental import pallas as pl
from jax.experimental.pallas import tpu as pltpu
from jax.experimental.pallas import tpu_sc as plsc

N = 10000
NP = 10240
E = 320000
EP = 327680
WORKERS = 32
BATCH = 128
TOTB = EP // BATCH
NB0 = 152
NB1 = 8
S = 8
ROWS_PER_TILE = NP // 16
GATHER_ONLY = False
SCATTER_ONLY = False


def _seg_sum_call(x3, src3, dst3, C, F):
    mesh = plsc.VectorSubcoreMesh(core_axis_name="c", subcore_axis_name="s")

    @functools.partial(
        pl.kernel,
        mesh=mesh,
        out_type=jax.ShapeDtypeStruct((2, C, NP, F), jnp.float32),
        scratch_types=[
            pltpu.VMEM((2, S, BATCH), jnp.int32),
            pltpu.VMEM((2, S, BATCH), jnp.int32),
            pltpu.VMEM((2 * BATCH, F), jnp.float32),
            pltpu.VMEM_SHARED((NP, F), jnp.float32),
            pltpu.SemaphoreType.DMA,
            pltpu.SemaphoreType.DMA,
            pltpu.SemaphoreType.DMA,
            pltpu.SemaphoreType.DMA,
            pltpu.SemaphoreType.DMA,
        ],
    )
    def seg_kernel(x3_hbm, src_hbm, dst_hbm, out_hbm,
                   isrc, idst, rows, agg_sh, sem_i, sg0, sg1, ss0, ss1):
        core = lax.axis_index("c")
        sub = lax.axis_index("s")
        row0 = sub * ROWS_PER_TILE
        sg = (sg0, sg1)
        ss = (ss0, ss1)
        base = jnp.where(core == 0, sub * NB0, 16 * NB0 + sub * NB1)
        nsb = jnp.where(core == 0, NB0 // S, NB1 // S)

        def slot(p):
            return rows.at[pl.ds(p * BATCH, BATCH)]

        def idx_desc(q, sb):
            return (pltpu.make_async_copy(
                        src_hbm.at[pl.ds(base + sb * S, S)], isrc.at[q], sem_i),
                    pltpu.make_async_copy(
                        dst_hbm.at[pl.ds(base + sb * S, S)], idst.at[q], sem_i))

        def g_desc(c, q, j, p):
            return pltpu.make_async_copy(
                x3_hbm.at[c].at[isrc.at[q, j]], slot(p), sg[p])

        def s_desc(q, j, p):
            return pltpu.make_async_copy(
                slot(p), agg_sh.at[idst.at[q, j]], ss[p])

        def chunk_body(c, _):
            def zrow(r, _):
                def zcol(k2, _):
                    rows[r, pl.ds(k2 * 16, 16)] = jnp.zeros((16,), jnp.float32)
                    return 0
                return lax.fori_loop(0, F // 16, zcol, 0)
            lax.fori_loop(0, 64, zrow, 0)
            zsrc = rows.at[pl.ds(0, 64)]
            for i in range(ROWS_PER_TILE // 64):
                pltpu.async_copy(zsrc, agg_sh.at[pl.ds(row0 + i * 64, 64)], ss0)
            for i in range(ROWS_PER_TILE // 64):
                pltpu.make_async_copy(
                    zsrc, agg_sh.at[pl.ds(row0 + i * 64, 64)], ss0).wait()
            plsc.subcore_barrier()

            @pl.when(nsb > 0)
            def _():
                for d in idx_desc(0, 0):
                    d.start()

            def sb_body(sb, _):
                q = sb % 2
                for j in range(S):
                    p = j % 2
                    if not GATHER_ONLY:
                        if j >= 2:
                            s_desc(q, j - 2, p).wait()
                        else:
                            @pl.when(sb > 0)
                            def _():
                                s_desc(1 - q, j + S - 2, p).wait()
                    if j == 0:
                        for d in idx_desc(q, sb):
                            d.wait()
                    if not SCATTER_ONLY:
                        g_desc(c, q, j, p).start()
                    if j >= 1:
                        if not SCATTER_ONLY:
                            g_desc(c, q, j - 1, 1 - p).wait()
                        if not GATHER_ONLY:
                            pltpu.async_copy(
                                slot(1 - p), agg_sh.at[idst.at[q, j - 1]],
                                ss[1 - p], add=True)
                    else:
                        @pl.when(sb > 0)
                        def _():
                            if not SCATTER_ONLY:
                                g_desc(c, 1 - q, S - 1, 1 - p).wait()
                            if not GATHER_ONLY:
                                pltpu.async_copy(
                                    slot(1 - p), agg_sh.at[idst.at[1 - q, S - 1]],
                                    ss[1 - p], add=True)
                        @pl.when(sb + 1 < nsb)
                        def _():
                            for d in idx_desc(1 - q, sb + 1):
                                d.start()
                return 0
            lax.fori_loop(0, nsb, sb_body, 0)

            @pl.when(nsb > 0)
            def _():
                qL = (nsb - 1) % 2
                pL = (S - 1) % 2
                if not SCATTER_ONLY:
                    g_desc(c, qL, S - 1, pL).wait()
                if not GATHER_ONLY:
                    pltpu.async_copy(slot(pL), agg_sh.at[idst.at[qL, S - 1]],
                                     ss[pL], add=True)
                    s_desc(qL, S - 2, 1 - pL).wait()
                    s_desc(qL, S - 1, pL).wait()
            plsc.subcore_barrier()

            pltpu.sync_copy(agg_sh.at[pl.ds(row0, ROWS_PER_TILE)],
                            out_hbm.at[core].at[c].at[pl.ds(row0, ROWS_PER_TILE)])
            plsc.subcore_barrier()
            return 0

        lax.fori_loop(0, C, chunk_body, 0)

    return seg_kernel(x3, src3, dst3)


RB = 256


def _mlp_call(x3, agg3, W1, b1, W2, b2, relu_out, out_chunks):
    C, _, F = x3.shape
    D, H = W1.shape
    H2 = W2.shape[1]
    C2 = out_chunks

    def body(x_ref, a_ref, w1_ref, b1_ref, w2_ref, b2_ref, o_ref):
        xb = jnp.concatenate(
            [x_ref[c] + a_ref[0, c] + a_ref[1, c] for c in range(C)], axis=1)
        h = jnp.dot(xb, w1_ref[...], preferred_element_type=jnp.float32)
        h = jnp.maximum(h + b1_ref[0], 0.0)
        y = jnp.dot(h, w2_ref[...], preferred_element_type=jnp.float32)
        y = y + b2_ref[0]
        if relu_out:
            y = jnp.maximum(y, 0.0)
        if C2 is None:
            o_ref[...] = y
        else:
            for c2 in range(C2):
                o_ref[c2] = y[:, c2 * 128:(c2 + 1) * 128]

    if C2 is None:
        out_shape = jax.ShapeDtypeStruct((NP, H2), jnp.float32)
        out_spec = pl.BlockSpec((RB, H2), lambda i: (i, 0))
    else:
        out_shape = jax.ShapeDtypeStruct((C2, NP, 128), jnp.float32)
        out_spec = pl.BlockSpec((C2, RB, 128), lambda i: (0, i, 0))

    return pl.pallas_call(
        body,
        grid=(NP // RB,),
        in_specs=[
            pl.BlockSpec((C, RB, F), lambda i: (0, i, 0)),
            pl.BlockSpec((2, C, RB, F), lambda i: (0, 0, i, 0)),
            pl.BlockSpec((D, H), lambda i: (0, 0)),
            pl.BlockSpec((1, H), lambda i: (0, 0)),
            pl.BlockSpec((H, H2), lambda i: (0, 0)),
            pl.BlockSpec((1, H2), lambda i: (0, 0)),
        ],
        out_specs=out_spec,
        out_shape=out_shape,
    )(x3, agg3, W1, b1.reshape(1, H), W2, b2.reshape(1, H2))


def _proj_call(x3, W, out_chunks):
    C, _, F = x3.shape
    D, H2 = W.shape
    C2 = out_chunks

    def body(x_ref, w_ref, o_ref):
        xb = jnp.concatenate([x_ref[c] for c in range(C)], axis=1)
        y = jnp.dot(xb, w_ref[...], preferred_element_type=jnp.float32)
        for c2 in range(C2):
            o_ref[c2] = y[:, c2 * 128:(c2 + 1) * 128]

    return pl.pallas_call(
        body,
        grid=(NP // RB,),
        in_specs=[
            pl.BlockSpec((C, RB, F), lambda i: (0, i, 0)),
            pl.BlockSpec((D, H2), lambda i: (0, 0)),
        ],
        out_specs=pl.BlockSpec((C2, RB, 128), lambda i: (0, i, 0)),
        out_shape=jax.ShapeDtypeStruct((C2, NP, 128), jnp.float32),
    )(x3, W)


def _gin_tail_call(u3, agg3, b1, W2, b2):
    C, _, F = u3.shape
    H = C * F
    H2 = W2.shape[1]

    def body(u_ref, a_ref, b1_ref, w2_ref, b2_ref, o_ref):
        h = jnp.concatenate(
            [u_ref[c] + a_ref[0, c] + a_ref[1, c] for c in range(C)], axis=1)
        h = jnp.maximum(h + b1_ref[0], 0.0)
        o_ref[...] = jnp.dot(
            h, w2_ref[...], preferred_element_type=jnp.float32) + b2_ref[0]

    return pl.pallas_call(
        body,
        grid=(NP // RB,),
        in_specs=[
            pl.BlockSpec((C, RB, F), lambda i: (0, i, 0)),
            pl.BlockSpec((2, C, RB, F), lambda i: (0, 0, i, 0)),
            pl.BlockSpec((1, H), lambda i: (0, 0)),
            pl.BlockSpec((H, H2), lambda i: (0, 0)),
            pl.BlockSpec((1, H2), lambda i: (0, 0)),
        ],
        out_specs=pl.BlockSpec((RB, H2), lambda i: (i, 0)),
        out_shape=jax.ShapeDtypeStruct((NP, H2), jnp.float32),
    )(u3, agg3, b1.reshape(1, H), W2, b2.reshape(1, H2))


def _heads_call(x, Wcat, bcat, eps):
    H2 = x.shape[1]

    def body(x_ref, w_ref, b_ref, e_ref, z_ref, m_ref, v_ref):
        y = jnp.dot(x_ref[...], w_ref[...],
                    preferred_element_type=jnp.float32) + b_ref[0]
        m = y[:, :128]
        v = y[:, 128:]
        m_ref[...] = m
        v_ref[...] = v
        z_ref[...] = m + v * e_ref[...]

    out_shape = [jax.ShapeDtypeStruct((NP, 128), jnp.float32)] * 3
    return pl.pallas_call(
        body,
        grid=(NP // RB,),
        in_specs=[
            pl.BlockSpec((RB, H2), lambda i: (i, 0)),
            pl.BlockSpec((H2, 256), lambda i: (0, 0)),
            pl.BlockSpec((1, 256), lambda i: (0, 0)),
            pl.BlockSpec((RB, 128), lambda i: (i, 0)),
        ],
        out_specs=[pl.BlockSpec((RB, 128), lambda i: (i, 0))] * 3,
        out_shape=out_shape,
    )(x, Wcat, bcat.reshape(1, 256), eps)


def _to_chunks(x, C, F):
    return jnp.transpose(x.reshape(NP, C, F), (1, 0, 2))


def kernel(eeg_nodes, eeg_idx,
           W1_0, b1_0, W2_0, b2_0,
           W1_1, b1_1, W2_1, b2_1,
           W1_2, b1_2, W2_2, b2_2,
           W1_3, b1_3, W2_3, b2_3,
           Wm, bm, Wv, bv):
    src = eeg_idx[0]
    dst = eeg_idx[1]
    pad_e = EP - E
    src_p = jnp.concatenate([src, jnp.zeros((pad_e,), jnp.int32)])
    dst_p = jnp.concatenate([dst, jnp.full((pad_e,), N, jnp.int32)])
    src3 = src_p.reshape(TOTB, BATCH)
    dst3 = dst_p.reshape(TOTB, BATCH)

    x0 = jnp.pad(eeg_nodes, ((0, NP - N), (0, 0)))
    x3_0 = x0[None]

    agg0 = _seg_sum_call(x3_0, src3, dst3, C=1, F=128)
    x1 = _mlp_call(x3_0, agg0, W1_0, b1_0, W2_0, b2_0,
                   relu_out=True, out_chunks=16)

    agg1 = _seg_sum_call(x1, src3, dst3, C=16, F=128)
    x2 = _mlp_call(x1, agg1, W1_1, b1_1, W2_1, b2_1,
                   relu_out=True, out_chunks=16)

    agg2 = _seg_sum_call(x2, src3, dst3, C=16, F=128)
    x3_ = _mlp_call(x2, agg2, W1_2, b1_2, W2_2, b2_2,
                    relu_out=True, out_chunks=16)

    u3 = _proj_call(x3_, W1_3, out_chunks=8)
    agg3 = _seg_sum_call(u3, src3, dst3, C=8, F=128)
    x4 = _gin_tail_call(u3, agg3, b1_3, W2_3, b2_3)

    Wcat = jnp.concatenate([Wm, Wv], axis=1)
    bcat = jnp.concatenate([bm, bv])
    epsn = jax.random.normal(jax.random.key(1234), (N, 128), jnp.float32)
    eps_p = jnp.pad(epsn, ((0, NP - N), (0, 0)))
    z, mean, var = _heads_call(x4, Wcat, bcat, eps_p)
    return (z[:N], mean[:N], var[:N])


# speedup vs baseline: 1.3940x; 1.0001x over previous
import functools

import jax
import jax.numpy as jnp
from jax import lax
from jax.experimental import pallas as pl
from jax.experimental.pallas import tpu as pltpu
from jax.experimental.pallas import tpu_sc as plsc

N = 10000
NP = 10240
E = 320000
EP = 327680
BATCH = 128
TOTB = EP // BATCH
NB0 = 152
NB1 = 8
S = 8
ROWS_PER_TILE = NP // 16


def _seg_sum_call(x3, src3, dst3, C, F):
    mesh = plsc.VectorSubcoreMesh(core_axis_name="c", subcore_axis_name="s")

    @functools.partial(
        pl.kernel,
        mesh=mesh,
        out_type=jax.ShapeDtypeStruct((2, C, NP, F), jnp.float32),
        scratch_types=[
            pltpu.VMEM((2, S, BATCH), jnp.int32),
            pltpu.VMEM((2, S, BATCH), jnp.int32),
            pltpu.VMEM((2 * BATCH, F), jnp.float32),
            pltpu.VMEM_SHARED((NP, F), jnp.float32),
            pltpu.SemaphoreType.DMA,
            pltpu.SemaphoreType.DMA,
            pltpu.SemaphoreType.DMA,
            pltpu.SemaphoreType.DMA,
            pltpu.SemaphoreType.DMA,
        ],
    )
    def seg_kernel(x3_hbm, src_hbm, dst_hbm, out_hbm,
                   isrc, idst, rows, agg_sh, sem_i, sg0, sg1, ss0, ss1):
        core = lax.axis_index("c")
        sub = lax.axis_index("s")
        row0 = sub * ROWS_PER_TILE
        sg = (sg0, sg1)
        ss = (ss0, ss1)
        base = jnp.where(core == 0, sub * NB0, 16 * NB0 + sub * NB1)
        nsb = jnp.where(core == 0, NB0 // S, NB1 // S)

        def slot(p):
            return rows.at[pl.ds(p * BATCH, BATCH)]

        def idx_desc(q, sb):
            return (pltpu.make_async_copy(
                        src_hbm.at[pl.ds(base + sb * S, S)], isrc.at[q], sem_i),
                    pltpu.make_async_copy(
                        dst_hbm.at[pl.ds(base + sb * S, S)], idst.at[q], sem_i))

        def g_desc(c, q, j, p):
            return pltpu.make_async_copy(
                x3_hbm.at[c].at[isrc.at[q, j]], slot(p), sg[p])

        def s_desc(q, j, p):
            return pltpu.make_async_copy(
                slot(p), agg_sh.at[idst.at[q, j]], ss[p])

        def chunk_body(c, _):
            def zrow(r, _):
                def zcol(k2, _):
                    rows[r, pl.ds(k2 * 16, 16)] = jnp.zeros((16,), jnp.float32)
                    return 0
                return lax.fori_loop(0, F // 16, zcol, 0)
            lax.fori_loop(0, 64, zrow, 0)
            zsrc = rows.at[pl.ds(0, 64)]
            for i in range(ROWS_PER_TILE // 64):
                pltpu.async_copy(zsrc, agg_sh.at[pl.ds(row0 + i * 64, 64)], ss0)
            for i in range(ROWS_PER_TILE // 64):
                pltpu.make_async_copy(
                    zsrc, agg_sh.at[pl.ds(row0 + i * 64, 64)], ss0).wait()
            plsc.subcore_barrier()

            @pl.when(nsb > 0)
            def _():
                for d in idx_desc(0, 0):
                    d.start()

            def sb_body(sb, _):
                q = sb % 2
                for j in range(S):
                    p = j % 2
                    if j >= 2:
                        s_desc(q, j - 2, p).wait()
                    else:
                        @pl.when(sb > 0)
                        def _():
                            s_desc(1 - q, j + S - 2, p).wait()
                    if j == 0:
                        for d in idx_desc(q, sb):
                            d.wait()
                    g_desc(c, q, j, p).start()
                    if j >= 1:
                        g_desc(c, q, j - 1, 1 - p).wait()
                        pltpu.async_copy(
                            slot(1 - p), agg_sh.at[idst.at[q, j - 1]],
                            ss[1 - p], add=True)
                    else:
                        @pl.when(sb > 0)
                        def _():
                            g_desc(c, 1 - q, S - 1, 1 - p).wait()
                            pltpu.async_copy(
                                slot(1 - p), agg_sh.at[idst.at[1 - q, S - 1]],
                                ss[1 - p], add=True)
                        @pl.when(sb + 1 < nsb)
                        def _():
                            for d in idx_desc(1 - q, sb + 1):
                                d.start()
                return 0
            lax.fori_loop(0, nsb, sb_body, 0)

            @pl.when(nsb > 0)
            def _():
                qL = (nsb - 1) % 2
                pL = (S - 1) % 2
                g_desc(c, qL, S - 1, pL).wait()
                pltpu.async_copy(slot(pL), agg_sh.at[idst.at[qL, S - 1]],
                                 ss[pL], add=True)
                s_desc(qL, S - 2, 1 - pL).wait()
                s_desc(qL, S - 1, pL).wait()
            plsc.subcore_barrier()

            pltpu.sync_copy(agg_sh.at[pl.ds(row0, ROWS_PER_TILE)],
                            out_hbm.at[core].at[c].at[pl.ds(row0, ROWS_PER_TILE)])
            plsc.subcore_barrier()
            return 0

        lax.fori_loop(0, C, chunk_body, 0)

    return seg_kernel(x3, src3, dst3)


RB = 256


def _mlp_call(x3, agg3, W1, b1, W2, b2, relu_out, out_chunks):
    C, _, F = x3.shape
    D, H = W1.shape
    H2 = W2.shape[1]
    C2 = out_chunks

    def body(x_ref, a_ref, w1_ref, b1_ref, w2_ref, b2_ref, o_ref):
        xb = jnp.concatenate(
            [x_ref[c] + a_ref[0, c] + a_ref[1, c] for c in range(C)], axis=1)
        h = jnp.dot(xb, w1_ref[...], preferred_element_type=jnp.float32)
        h = jnp.maximum(h + b1_ref[0], 0.0)
        y = jnp.dot(h, w2_ref[...], preferred_element_type=jnp.float32)
        y = y + b2_ref[0]
        if relu_out:
            y = jnp.maximum(y, 0.0)
        if C2 is None:
            o_ref[...] = y
        else:
            for c2 in range(C2):
                o_ref[c2] = y[:, c2 * 128:(c2 + 1) * 128]

    if C2 is None:
        out_shape = jax.ShapeDtypeStruct((NP, H2), jnp.float32)
        out_spec = pl.BlockSpec((RB, H2), lambda i: (i, 0))
    else:
        out_shape = jax.ShapeDtypeStruct((C2, NP, 128), jnp.float32)
        out_spec = pl.BlockSpec((C2, RB, 128), lambda i: (0, i, 0))

    return pl.pallas_call(
        body,
        grid=(NP // RB,),
        in_specs=[
            pl.BlockSpec((C, RB, F), lambda i: (0, i, 0)),
            pl.BlockSpec((2, C, RB, F), lambda i: (0, 0, i, 0)),
            pl.BlockSpec((D, H), lambda i: (0, 0)),
            pl.BlockSpec((1, H), lambda i: (0, 0)),
            pl.BlockSpec((H, H2), lambda i: (0, 0)),
            pl.BlockSpec((1, H2), lambda i: (0, 0)),
        ],
        out_specs=out_spec,
        out_shape=out_shape,
    )(x3, agg3, W1, b1.reshape(1, H), W2, b2.reshape(1, H2))


def _proj_call(x3, W, out_chunks):
    C, _, F = x3.shape
    D, H2 = W.shape
    C2 = out_chunks

    def body(x_ref, w_ref, o_ref):
        xb = jnp.concatenate([x_ref[c] for c in range(C)], axis=1)
        y = jnp.dot(xb, w_ref[...], preferred_element_type=jnp.float32)
        for c2 in range(C2):
            o_ref[c2] = y[:, c2 * 128:(c2 + 1) * 128]

    return pl.pallas_call(
        body,
        grid=(NP // RB,),
        in_specs=[
            pl.BlockSpec((C, RB, F), lambda i: (0, i, 0)),
            pl.BlockSpec((D, H2), lambda i: (0, 0)),
        ],
        out_specs=pl.BlockSpec((C2, RB, 128), lambda i: (0, i, 0)),
        out_shape=jax.ShapeDtypeStruct((C2, NP, 128), jnp.float32),
    )(x3, W)


def _gin_tail_call(u3, agg3, b1, W2, b2):
    C, _, F = u3.shape
    H = C * F
    H2 = W2.shape[1]

    def body(u_ref, a_ref, b1_ref, w2_ref, b2_ref, o_ref):
        h = jnp.concatenate(
            [u_ref[c] + a_ref[0, c] + a_ref[1, c] for c in range(C)], axis=1)
        h = jnp.maximum(h + b1_ref[0], 0.0)
        o_ref[...] = jnp.dot(
            h, w2_ref[...], preferred_element_type=jnp.float32) + b2_ref[0]

    return pl.pallas_call(
        body,
        grid=(NP // RB,),
        in_specs=[
            pl.BlockSpec((C, RB, F), lambda i: (0, i, 0)),
            pl.BlockSpec((2, C, RB, F), lambda i: (0, 0, i, 0)),
            pl.BlockSpec((1, H), lambda i: (0, 0)),
            pl.BlockSpec((H, H2), lambda i: (0, 0)),
            pl.BlockSpec((1, H2), lambda i: (0, 0)),
        ],
        out_specs=pl.BlockSpec((RB, H2), lambda i: (i, 0)),
        out_shape=jax.ShapeDtypeStruct((NP, H2), jnp.float32),
    )(u3, agg3, b1.reshape(1, H), W2, b2.reshape(1, H2))


def _heads_call(x, Wcat, bcat, eps):
    H2 = x.shape[1]

    def body(x_ref, w_ref, b_ref, e_ref, z_ref, m_ref, v_ref):
        y = jnp.dot(x_ref[...], w_ref[...],
                    preferred_element_type=jnp.float32) + b_ref[0]
        m = y[:, :128]
        v = y[:, 128:]
        m_ref[...] = m
        v_ref[...] = v
        z_ref[...] = m + v * e_ref[...]

    out_shape = [jax.ShapeDtypeStruct((NP, 128), jnp.float32)] * 3
    return pl.pallas_call(
        body,
        grid=(NP // RB,),
        in_specs=[
            pl.BlockSpec((RB, H2), lambda i: (i, 0)),
            pl.BlockSpec((H2, 256), lambda i: (0, 0)),
            pl.BlockSpec((1, 256), lambda i: (0, 0)),
            pl.BlockSpec((RB, 128), lambda i: (i, 0)),
        ],
        out_specs=[pl.BlockSpec((RB, 128), lambda i: (i, 0))] * 3,
        out_shape=out_shape,
    )(x, Wcat, bcat.reshape(1, 256), eps)


def kernel(eeg_nodes, eeg_idx,
           W1_0, b1_0, W2_0, b2_0,
           W1_1, b1_1, W2_1, b2_1,
           W1_2, b1_2, W2_2, b2_2,
           W1_3, b1_3, W2_3, b2_3,
           Wm, bm, Wv, bv):
    src = eeg_idx[0]
    dst = eeg_idx[1]
    pad_e = EP - E
    src_p = jnp.concatenate([src, jnp.zeros((pad_e,), jnp.int32)])
    dst_p = jnp.concatenate([dst, jnp.full((pad_e,), N, jnp.int32)])
    src3 = src_p.reshape(TOTB, BATCH)
    dst3 = dst_p.reshape(TOTB, BATCH)

    x0 = jnp.pad(eeg_nodes, ((0, NP - N), (0, 0)))
    x3_0 = x0[None]

    agg0 = _seg_sum_call(x3_0, src3, dst3, C=1, F=128)
    x1 = _mlp_call(x3_0, agg0, W1_0, b1_0, W2_0, b2_0,
                   relu_out=True, out_chunks=16)

    agg1 = _seg_sum_call(x1, src3, dst3, C=16, F=128)
    x2 = _mlp_call(x1, agg1, W1_1, b1_1, W2_1, b2_1,
                   relu_out=True, out_chunks=16)

    agg2 = _seg_sum_call(x2, src3, dst3, C=16, F=128)
    x3_ = _mlp_call(x2, agg2, W1_2, b1_2, W2_2, b2_2,
                    relu_out=True, out_chunks=16)

    u3 = _proj_call(x3_, W1_3, out_chunks=8)
    agg3 = _seg_sum_call(u3, src3, dst3, C=8, F=128)
    x4 = _gin_tail_call(u3, agg3, b1_3, W2_3, b2_3)

    Wcat = jnp.concatenate([Wm, Wv], axis=1)
    bcat = jnp.concatenate([bm, bv])
    epsn = jax.random.normal(jax.random.key(1234), (N, 128), jnp.float32)
    eps_p = jnp.pad(epsn, ((0, NP - N), (0, 0)))
    z, mean, var = _heads_call(x4, Wcat, bcat, eps_p)
    return (z[:N], mean[:N], var[:N])
